# baseline (speedup 1.0000x reference)
import jax
import jax.numpy as jnp
from jax.experimental import pallas as pl

N = 32768
D = 256
H = 128
M = 16
BLK = 2048


def _fused_body(x_ref, idx_ref, w1_ref, b1_ref, w2_ref, b2_ref, out_ref):
    i = pl.program_id(0)

    @pl.when(i == 0)
    def _init():
        out_ref[...] = jnp.zeros_like(out_ref)

    x = x_ref[...].astype(jnp.bfloat16)
    h = jnp.dot(x, w1_ref[...].astype(jnp.bfloat16),
                preferred_element_type=jnp.float32)
    h = h + b1_ref[...]
    h = h * jax.nn.sigmoid(h)
    y = jnp.dot(h, w2_ref[...], preferred_element_type=jnp.float32)
    y = y + b2_ref[...]

    idx = idx_ref[...]
    sel = (idx == jax.lax.broadcasted_iota(jnp.int32, (1, M), 1)).astype(
        jnp.float32)
    partial = jax.lax.dot_general(
        y, sel, (((0,), (0,)), ((), ())),
        preferred_element_type=jnp.float32)
    out_ref[...] += partial


def kernel(scalar_representation, idx_m, W1, b1, W2, b2):
    idx2d = idx_m.astype(jnp.int32).reshape(N, 1)
    out = pl.pallas_call(
        _fused_body,
        grid=(N // BLK,),
        in_specs=[
            pl.BlockSpec((BLK, D), lambda i: (i, 0)),
            pl.BlockSpec((BLK, 1), lambda i: (i, 0)),
            pl.BlockSpec((D, H), lambda i: (0, 0)),
            pl.BlockSpec((1, H), lambda i: (0, 0)),
            pl.BlockSpec((H, 1), lambda i: (0, 0)),
            pl.BlockSpec((1, 1), lambda i: (0, 0)),
        ],
        out_specs=pl.BlockSpec((1, M), lambda i: (0, 0)),
        out_shape=jax.ShapeDtypeStruct((1, M), jnp.float32),
    )(scalar_representation, idx2d, W1, b1.reshape(1, H), W2,
      b2.reshape(1, 1))
    return out.reshape(M)


# speedup vs baseline: 1.1829x; 1.1829x over previous
import jax
import jax.numpy as jnp
from jax.experimental import pallas as pl

N = 32768
D = 256
H = 128
M = 16
BLK = 8192


def _fused_body(x_ref, idx_ref, w1_ref, b1_ref, w2_ref, b2_ref, out_ref):
    i = pl.program_id(0)

    @pl.when(i == 0)
    def _init():
        out_ref[...] = jnp.zeros_like(out_ref)

    x = x_ref[...].astype(jnp.bfloat16)
    h = jnp.dot(x, w1_ref[...].astype(jnp.bfloat16),
                preferred_element_type=jnp.float32)
    h = h + b1_ref[...]
    h = h * jax.nn.sigmoid(h)
    y = jnp.dot(h, w2_ref[...], preferred_element_type=jnp.float32)
    y = y + b2_ref[...]

    idx = idx_ref[...]
    sel = (idx == jax.lax.broadcasted_iota(jnp.int32, (1, M), 1)).astype(
        jnp.float32)
    partial = jax.lax.dot_general(
        y, sel, (((0,), (0,)), ((), ())),
        preferred_element_type=jnp.float32)
    out_ref[...] += partial


def kernel(scalar_representation, idx_m, W1, b1, W2, b2):
    idx2d = idx_m.astype(jnp.int32).reshape(N, 1)
    out = pl.pallas_call(
        _fused_body,
        grid=(N // BLK,),
        in_specs=[
            pl.BlockSpec((BLK, D), lambda i: (i, 0)),
            pl.BlockSpec((BLK, 1), lambda i: (i, 0)),
            pl.BlockSpec((D, H), lambda i: (0, 0)),
            pl.BlockSpec((1, H), lambda i: (0, 0)),
            pl.BlockSpec((H, 1), lambda i: (0, 0)),
            pl.BlockSpec((1, 1), lambda i: (0, 0)),
        ],
        out_specs=pl.BlockSpec((1, M), lambda i: (0, 0)),
        out_shape=jax.ShapeDtypeStruct((1, M), jnp.float32),
    )(scalar_representation, idx2d, W1, b1.reshape(1, H), W2,
      b2.reshape(1, 1))
    return out.reshape(M)
